# baseline (speedup 1.0000x reference)
import functools

import jax
import jax.numpy as jnp
from jax import lax
from jax.experimental import pallas as pl
from jax.experimental.pallas import tpu as pltpu
from jax.experimental.pallas import tpu_sc as plsc

N = 10000
E = 320000
D = 128
G = 128
DOUT = 16

NC = 2
NS = 16
NW = NC * NS
CHUNK = 128
E_PAD = 327680
C0_CHUNKS = 160
STAGES = 2
SPS = C0_CHUNKS // STAGES
DEG0 = 96
DEG1 = 64
N_PAD = 10240
RPT = N_PAD // NS

_mesh = plsc.VectorSubcoreMesh(core_axis_name="core", subcore_axis_name="subcore")


@functools.partial(
    pl.kernel,
    out_type=jax.ShapeDtypeStruct((NC, N_PAD, D), jnp.float32),
    mesh=_mesh,
    scratch_types=[
        pltpu.VMEM_SHARED((N_PAD, D), jnp.float32),
        pltpu.VMEM((DEG0, CHUNK), jnp.int32),
        pltpu.VMEM((CHUNK, D), jnp.float32),
        pltpu.SemaphoreType.DMA,
        pltpu.SemaphoreType.DMA,
    ],
)
def _sc_degree(dst_hbm, zeros_hbm, ones_hbm, out_hbm, acc, dst_all, ones_v,
               ssem0, ssem1):
    c = lax.axis_index("core")
    s = lax.axis_index("subcore")
    w = c * NS + s
    r0 = s * RPT
    pltpu.sync_copy(zeros_hbm.at[pl.ds(r0, RPT)], acc.at[pl.ds(r0, RPT)])
    pltpu.sync_copy(dst_hbm.at[w], dst_all)
    pltpu.sync_copy(ones_hbm, ones_v)
    plsc.subcore_barrier()

    GRP = 8
    my_groups = jnp.where(c == 0, DEG0 // GRP, DEG1 // GRP)

    @pl.loop(0, my_groups)
    def _(j):
        for t in range(GRP):
            sem = ssem0 if t % 2 == 0 else ssem1
            pltpu.async_copy(ones_v, acc.at[dst_all.at[GRP * j + t]], sem,
                             add=True)
        for t in range(GRP):
            sem = ssem0 if t % 2 == 0 else ssem1
            pltpu.make_async_copy(ones_v, acc.at[dst_all.at[GRP * j + t]],
                                  sem).wait()

    plsc.subcore_barrier()
    pltpu.sync_copy(acc.at[pl.ds(r0, RPT)], out_hbm.at[c, pl.ds(r0, RPT)])


@functools.partial(
    pl.kernel,
    out_type=jax.ShapeDtypeStruct((NC, N_PAD, D), jnp.float32),
    mesh=_mesh,
    scratch_types=[
        pltpu.VMEM_SHARED((N_PAD, D), jnp.float32),
        pltpu.VMEM((SPS, CHUNK), jnp.int32),
        pltpu.VMEM((CHUNK,), jnp.int32),
        pltpu.VMEM((CHUNK,), jnp.int32),
        pltpu.VMEM((CHUNK, D), jnp.float32),
        pltpu.VMEM((CHUNK, D), jnp.float32),
        pltpu.SemaphoreType.DMA,
        pltpu.SemaphoreType.DMA,
        pltpu.SemaphoreType.DMA,
        pltpu.SemaphoreType.DMA,
        pltpu.SemaphoreType.DMA,
        pltpu.SemaphoreType.DMA,
    ],
)
def _sc_scatter(hs_hbm, src_hbm, dst_hbm, zeros_hbm, out_hbm,
                acc, src_all, di0, di1, rows0, rows1,
                gsem0, gsem1, ssem0, ssem1, dsem0, dsem1):
    c = lax.axis_index("core")
    s = lax.axis_index("subcore")
    r0 = s * RPT
    pltpu.sync_copy(zeros_hbm.at[pl.ds(r0, RPT)], acc.at[pl.ds(r0, RPT)])
    plsc.subcore_barrier()

    @pl.when(c == 0)
    def _():
        for st in range(STAGES):
            g0 = st * SPS
            pltpu.sync_copy(src_hbm.at[s, pl.ds(g0, SPS)], src_all)
            pltpu.async_copy(hs_hbm.at[src_all.at[0]], rows0, gsem0)
            pltpu.async_copy(hs_hbm.at[src_all.at[1]], rows1, gsem1)
            pltpu.async_copy(dst_hbm.at[s, g0], di0, dsem0)
            pltpu.async_copy(dst_hbm.at[s, g0 + 1], di1, dsem1)

            HALF = SPS // 2

            @pl.loop(0, HALF)
            def _(j):
                l0 = 2 * j
                l1 = l0 + 1
                pltpu.make_async_copy(
                    hs_hbm.at[src_all.at[l0]], rows0, gsem0).wait()
                pltpu.make_async_copy(
                    dst_hbm.at[s, g0 + l0], di0, dsem0).wait()
                pltpu.async_copy(rows0, acc.at[di0], ssem0, add=True)
                pltpu.make_async_copy(
                    hs_hbm.at[src_all.at[l1]], rows1, gsem1).wait()
                pltpu.make_async_copy(
                    dst_hbm.at[s, g0 + l1], di1, dsem1).wait()
                pltpu.async_copy(rows1, acc.at[di1], ssem1, add=True)
                pltpu.make_async_copy(rows0, acc.at[di0], ssem0).wait()
                pltpu.make_async_copy(rows1, acc.at[di1], ssem1).wait()

                @pl.when(j < HALF - 1)
                def _():
                    pltpu.async_copy(hs_hbm.at[src_all.at[l0 + 2]], rows0,
                                     gsem0)
                    pltpu.async_copy(hs_hbm.at[src_all.at[l1 + 2]], rows1,
                                     gsem1)
                    pltpu.async_copy(dst_hbm.at[s, g0 + l0 + 2], di0, dsem0)
                    pltpu.async_copy(dst_hbm.at[s, g0 + l1 + 2], di1, dsem1)

    plsc.subcore_barrier()
    pltpu.sync_copy(acc.at[pl.ds(r0, RPT)], out_hbm.at[c, pl.ds(r0, RPT)])


def _tc_matmul_body(x_ref, w_ref, o_ref):
    o_ref[...] = jnp.dot(x_ref[...], w_ref[...],
                         preferred_element_type=jnp.float32)


def _tc_scale_body(deg_ref, h_ref, hs_ref, dinv_ref):
    deg = deg_ref[0, :N, 0:1] + deg_ref[1, :N, 0:1] + 1.0
    dinv = lax.rsqrt(deg)
    dinv_ref[...] = dinv
    hs_ref[...] = h_ref[...] * dinv


def _tc_mid_body(acc_ref, hs_ref, dinv_ref, b_ref, w_ref, o_ref):
    a = acc_ref[0, :N, :] + acc_ref[1, :N, :] + hs_ref[...]
    dinv = dinv_ref[...]
    h = jnp.maximum(a * dinv + b_ref[...], 0.0)
    o_ref[...] = jnp.dot(h, w_ref[...],
                         preferred_element_type=jnp.float32) * dinv


def _tc_final_body(acc_ref, hs_ref, dinv_ref, b_ref, batch_ref, wm_ref,
                   bm_ref, o_ref):
    a = acc_ref[0, :N, :] + acc_ref[1, :N, :] + hs_ref[...]
    h3 = jnp.maximum(a * dinv_ref[...] + b_ref[...], 0.0)
    gid = lax.broadcasted_iota(jnp.int32, (G, N), 0)
    onehot = (batch_ref[...] == gid).astype(jnp.float32)
    pooled = jnp.dot(onehot, h3, preferred_element_type=jnp.float32)
    o_ref[...] = jnp.dot(pooled, wm_ref[...],
                         preferred_element_type=jnp.float32) + bm_ref[...]


def kernel(x, edge_index, batch, W0, b0, W1, b1, W2, b2, Wm, bm):
    src = edge_index[0].astype(jnp.int32)
    dst = edge_index[1].astype(jnp.int32)
    npad = E_PAD - E
    src_p = jnp.concatenate([src, jnp.zeros((npad,), jnp.int32)])
    dst_p = jnp.concatenate([dst, jnp.full((npad,), N, jnp.int32)])

    src_f = src_p.reshape(NS, C0_CHUNKS, CHUNK)
    dst_f = dst_p.reshape(NS, C0_CHUNKS, CHUNK)

    e0 = NS * DEG0 * CHUNK
    d0 = dst_p[:e0].reshape(NS, DEG0, CHUNK)
    d1 = dst_p[e0:].reshape(NS, DEG1, CHUNK)
    d1 = jnp.concatenate(
        [d1, jnp.full((NS, DEG0 - DEG1, CHUNK), N, jnp.int32)], axis=1)
    dst_deg = jnp.concatenate([d0, d1], axis=0)
    batch_row = batch.astype(jnp.int32).reshape(1, N)

    onesD = jnp.ones((CHUNK, D), jnp.float32)
    zerosD = jnp.zeros((N_PAD, D), jnp.float32)

    deg2 = _sc_degree(dst_deg, zerosD, onesD)

    h0 = pl.pallas_call(
        _tc_matmul_body,
        out_shape=jax.ShapeDtypeStruct((N, D), jnp.float32),
    )(x, W0)

    hs0, dinv = pl.pallas_call(
        _tc_scale_body,
        out_shape=(jax.ShapeDtypeStruct((N, D), jnp.float32),
                   jax.ShapeDtypeStruct((N, 1), jnp.float32)),
    )(deg2, h0)

    hs = hs0
    for b_l, W_next in ((b0, W1), (b1, W2)):
        acc = _sc_scatter(hs, src_f, dst_f, zerosD)
        hs = pl.pallas_call(
            _tc_mid_body,
            out_shape=jax.ShapeDtypeStruct((N, D), jnp.float32),
        )(acc, hs, dinv, b_l.reshape(1, D), W_next)

    acc = _sc_scatter(hs, src_f, dst_f, zerosD)
    out = pl.pallas_call(
        _tc_final_body,
        out_shape=jax.ShapeDtypeStruct((G, DOUT), jnp.float32),
    )(acc, hs, dinv, b2.reshape(1, D), batch_row, Wm, bm.reshape(1, DOUT))
    return out


# speedup vs baseline: 1.5317x; 1.5317x over previous
import functools

import jax
import jax.numpy as jnp
from jax import lax
from jax.experimental import pallas as pl
from jax.experimental.pallas import tpu as pltpu
from jax.experimental.pallas import tpu_sc as plsc

N = 10000
E = 320000
D = 128
G = 128
DOUT = 16

NC = 2
NS = 16
NW = NC * NS
CHUNK = 128
E_PAD = 327680
IT0 = 120
IT1 = 40
ITMAX = 120
N_PAD = 10240
RPT = N_PAD // NS

_mesh = plsc.VectorSubcoreMesh(core_axis_name="core", subcore_axis_name="subcore")


@functools.partial(
    pl.kernel,
    out_type=jax.ShapeDtypeStruct((NC, N_PAD, D), jnp.float32),
    mesh=_mesh,
    scratch_types=[
        pltpu.VMEM_SHARED((N_PAD, D), jnp.float32),
        pltpu.VMEM((ITMAX, CHUNK), jnp.int32),
        pltpu.VMEM((CHUNK, D), jnp.float32),
        pltpu.SemaphoreType.DMA,
        pltpu.SemaphoreType.DMA,
    ],
)
def _sc_degree(dst_hbm, zeros_hbm, ones_hbm, out_hbm, acc, dst_all, ones_v,
               ssem0, ssem1):
    c = lax.axis_index("core")
    s = lax.axis_index("subcore")
    w = c * NS + s
    r0 = s * RPT
    pltpu.sync_copy(zeros_hbm.at[pl.ds(r0, RPT)], acc.at[pl.ds(r0, RPT)])
    pltpu.sync_copy(dst_hbm.at[w], dst_all)
    pltpu.sync_copy(ones_hbm, ones_v)
    plsc.subcore_barrier()

    GRP = 8
    my_groups = jnp.where(c == 0, IT0 // GRP, IT1 // GRP)

    @pl.loop(0, my_groups)
    def _(j):
        for t in range(GRP):
            sem = ssem0 if t % 2 == 0 else ssem1
            pltpu.async_copy(ones_v, acc.at[dst_all.at[GRP * j + t]], sem,
                             add=True)
        for t in range(GRP):
            sem = ssem0 if t % 2 == 0 else ssem1
            pltpu.make_async_copy(ones_v, acc.at[dst_all.at[GRP * j + t]],
                                  sem).wait()

    plsc.subcore_barrier()
    pltpu.sync_copy(acc.at[pl.ds(r0, RPT)], out_hbm.at[c, pl.ds(r0, RPT)])


@functools.partial(
    pl.kernel,
    out_type=jax.ShapeDtypeStruct((NC, N_PAD, D), jnp.float32),
    mesh=_mesh,
    scratch_types=[
        pltpu.VMEM_SHARED((N_PAD, D), jnp.float32),
        pltpu.VMEM((ITMAX, CHUNK), jnp.int32),
        pltpu.VMEM((CHUNK,), jnp.int32),
        pltpu.VMEM((CHUNK,), jnp.int32),
        pltpu.VMEM((CHUNK, D), jnp.float32),
        pltpu.VMEM((CHUNK, D), jnp.float32),
        pltpu.SemaphoreType.DMA,
        pltpu.SemaphoreType.DMA,
        pltpu.SemaphoreType.DMA,
        pltpu.SemaphoreType.DMA,
        pltpu.SemaphoreType.DMA,
        pltpu.SemaphoreType.DMA,
    ],
)
def _sc_scatter(hs_hbm, src_hbm, dst_hbm, zeros_hbm, out_hbm,
                acc, src_all, di0, di1, rows0, rows1,
                gsem0, gsem1, ssem0, ssem1, dsem0, dsem1):
    c = lax.axis_index("core")
    s = lax.axis_index("subcore")
    w = c * NS + s
    r0 = s * RPT
    pltpu.sync_copy(zeros_hbm.at[pl.ds(r0, RPT)], acc.at[pl.ds(r0, RPT)])
    pltpu.sync_copy(src_hbm.at[w], src_all)
    plsc.subcore_barrier()

    my_half = jnp.where(c == 0, IT0 // 2, IT1 // 2)
    pltpu.async_copy(hs_hbm.at[src_all.at[0]], rows0, gsem0)
    pltpu.async_copy(hs_hbm.at[src_all.at[1]], rows1, gsem1)
    pltpu.async_copy(dst_hbm.at[w, 0], di0, dsem0)
    pltpu.async_copy(dst_hbm.at[w, 1], di1, dsem1)

    @pl.loop(0, my_half)
    def _(j):
        i0 = 2 * j
        i1 = i0 + 1
        pltpu.make_async_copy(hs_hbm.at[src_all.at[i0]], rows0, gsem0).wait()
        pltpu.make_async_copy(dst_hbm.at[w, i0], di0, dsem0).wait()
        pltpu.async_copy(rows0, acc.at[di0], ssem0, add=True)
        pltpu.make_async_copy(hs_hbm.at[src_all.at[i1]], rows1, gsem1).wait()
        pltpu.make_async_copy(dst_hbm.at[w, i1], di1, dsem1).wait()
        pltpu.async_copy(rows1, acc.at[di1], ssem1, add=True)
        pltpu.make_async_copy(rows0, acc.at[di0], ssem0).wait()
        pltpu.make_async_copy(rows1, acc.at[di1], ssem1).wait()

        @pl.when(j < my_half - 1)
        def _():
            pltpu.async_copy(hs_hbm.at[src_all.at[i0 + 2]], rows0, gsem0)
            pltpu.async_copy(hs_hbm.at[src_all.at[i1 + 2]], rows1, gsem1)
            pltpu.async_copy(dst_hbm.at[w, i0 + 2], di0, dsem0)
            pltpu.async_copy(dst_hbm.at[w, i1 + 2], di1, dsem1)

    plsc.subcore_barrier()
    pltpu.sync_copy(acc.at[pl.ds(r0, RPT)], out_hbm.at[c, pl.ds(r0, RPT)])


def _tc_matmul_body(x_ref, w_ref, o_ref):
    o_ref[...] = jnp.dot(x_ref[...], w_ref[...],
                         preferred_element_type=jnp.float32)


def _tc_scale_body(deg_ref, h_ref, hs_ref, dinv_ref):
    deg = deg_ref[0, :N, 0:1] + deg_ref[1, :N, 0:1] + 1.0
    dinv = lax.rsqrt(deg)
    dinv_ref[...] = dinv
    hs_ref[...] = h_ref[...] * dinv


def _tc_mid_body(acc_ref, hs_ref, dinv_ref, b_ref, w_ref, o_ref):
    a = acc_ref[0, :N, :] + acc_ref[1, :N, :] + hs_ref[...]
    dinv = dinv_ref[...]
    h = jnp.maximum(a * dinv + b_ref[...], 0.0)
    o_ref[...] = jnp.dot(h, w_ref[...],
                         preferred_element_type=jnp.float32) * dinv


def _tc_final_body(acc_ref, hs_ref, dinv_ref, b_ref, batch_ref, wm_ref,
                   bm_ref, o_ref):
    a = acc_ref[0, :N, :] + acc_ref[1, :N, :] + hs_ref[...]
    h3 = jnp.maximum(a * dinv_ref[...] + b_ref[...], 0.0)
    gid = lax.broadcasted_iota(jnp.int32, (G, N), 0)
    onehot = (batch_ref[...] == gid).astype(jnp.float32)
    pooled = jnp.dot(onehot, h3, preferred_element_type=jnp.float32)
    o_ref[...] = jnp.dot(pooled, wm_ref[...],
                         preferred_element_type=jnp.float32) + bm_ref[...]


def kernel(x, edge_index, batch, W0, b0, W1, b1, W2, b2, Wm, bm):
    src = edge_index[0].astype(jnp.int32)
    dst = edge_index[1].astype(jnp.int32)
    npad = E_PAD - E
    src_p = jnp.concatenate([src, jnp.zeros((npad,), jnp.int32)])
    dst_p = jnp.concatenate([dst, jnp.full((npad,), N, jnp.int32)])

    def _layout(e):
        e0 = NS * IT0 * CHUNK

        def _pad(p, it):
            if it == ITMAX:
                return p
            return jnp.concatenate(
                [p, jnp.zeros((NS, ITMAX - it, CHUNK), jnp.int32)], axis=1)

        p0 = _pad(e[:e0].reshape(NS, IT0, CHUNK), IT0)
        p1 = _pad(e[e0:].reshape(NS, IT1, CHUNK), IT1)
        return jnp.concatenate([p0, p1], axis=0)

    src_p = _layout(src_p)
    dst_p = _layout(dst_p)
    batch_row = batch.astype(jnp.int32).reshape(1, N)

    onesD = jnp.ones((CHUNK, D), jnp.float32)
    zerosD = jnp.zeros((N_PAD, D), jnp.float32)

    deg2 = _sc_degree(dst_p, zerosD, onesD)

    h0 = pl.pallas_call(
        _tc_matmul_body,
        out_shape=jax.ShapeDtypeStruct((N, D), jnp.float32),
    )(x, W0)

    hs0, dinv = pl.pallas_call(
        _tc_scale_body,
        out_shape=(jax.ShapeDtypeStruct((N, D), jnp.float32),
                   jax.ShapeDtypeStruct((N, 1), jnp.float32)),
    )(deg2, h0)

    hs = hs0
    for b_l, W_next in ((b0, W1), (b1, W2)):
        acc = _sc_scatter(hs, src_p, dst_p, zerosD)
        hs = pl.pallas_call(
            _tc_mid_body,
            out_shape=jax.ShapeDtypeStruct((N, D), jnp.float32),
        )(acc, hs, dinv, b_l.reshape(1, D), W_next)

    acc = _sc_scatter(hs, src_p, dst_p, zerosD)
    out = pl.pallas_call(
        _tc_final_body,
        out_shape=jax.ShapeDtypeStruct((G, DOUT), jnp.float32),
    )(acc, hs, dinv, b2.reshape(1, D), batch_row, Wm, bm.reshape(1, DOUT))
    return out
